# Initial kernel scaffold; baseline (speedup 1.0000x reference)
import jax
import jax.numpy as jnp
from jax.experimental import pallas as pl


def kernel(x):
    raise NotImplementedError("write your pallas kernel here")


# speedup vs baseline: 1.9536x; 1.9536x over previous
import functools

import jax
import jax.numpy as jnp
from jax import lax
from jax.experimental import pallas as pl
from jax.experimental.pallas import tpu as pltpu
from jax.experimental.pallas import tpu_sc as plsc

ROWS = 128
COLS = 32768
K = 8
L = 16
NC, NS = 2, 16
NW = NC * NS
RPW = ROWS // NW
CHUNKS = COLS // L
UNROLL = 8

_mesh = plsc.VectorSubcoreMesh(
    core_axis_name="c", subcore_axis_name="s", num_cores=NC, num_subcores=NS
)


@functools.partial(
    pl.kernel,
    out_type=jax.ShapeDtypeStruct((ROWS, L), jnp.float32),
    mesh=_mesh,
    scratch_types=[
        pltpu.VMEM((2, COLS), jnp.float32),
        pltpu.VMEM((RPW, L), jnp.float32),
        pltpu.SemaphoreType.DMA,
        pltpu.SemaphoreType.DMA,
    ],
    compiler_params=pltpu.CompilerParams(needs_layout_passes=False),
)
def _topk_sc(x_hbm, out_hbm, buf, obuf, sem0, sem1):
    wid = lax.axis_index("s") * NC + lax.axis_index("c")
    base = wid * RPW
    sems = (sem0, sem1)
    copies = [None, None]
    copies[0] = pltpu.async_copy(x_hbm.at[base], buf.at[0], sems[0])
    for r in range(RPW):
        slot = r % 2
        if r + 1 < RPW:
            copies[1 - slot] = pltpu.async_copy(
                x_hbm.at[base + r + 1], buf.at[1 - slot], sems[1 - slot]
            )
        copies[slot].wait()

        ninf = jnp.full((L,), -jnp.inf, dtype=jnp.float32)
        regs = (ninf,) * K

        def body(i, regs, slot=slot):
            regs = list(regs)
            for u in range(UNROLL):
                v = buf[slot, pl.ds((i * UNROLL + u) * L, L)]
                for j in range(K):
                    hi = jnp.maximum(regs[j], v)
                    v = jnp.minimum(regs[j], v)
                    regs[j] = hi
            return tuple(regs)

        regs = lax.fori_loop(0, CHUNKS // UNROLL, body, regs)

        best, _ = plsc.sort_key_val(regs[0], regs[0], descending=True)
        for j in range(1, K):
            asc, _ = plsc.sort_key_val(regs[j], regs[j], descending=False)
            best = jnp.maximum(best, asc)
            best, _ = plsc.sort_key_val(best, best, descending=True)
        obuf[r] = best
    pltpu.sync_copy(obuf, out_hbm.at[pl.ds(base, RPW)])


def kernel(x):
    return _topk_sc(x)[:, :K]


# speedup vs baseline: 2.3955x; 1.2262x over previous
import functools

import jax
import jax.numpy as jnp
from jax import lax
from jax.experimental import pallas as pl
from jax.experimental.pallas import tpu as pltpu
from jax.experimental.pallas import tpu_sc as plsc

ROWS = 128
COLS = 32768
K = 8
L = 16
NC, NS = 2, 16
NW = NC * NS
RPW = ROWS // NW
BLK = 8
NBLK = COLS // (BLK * L)
BLK_UNROLL = 2

SORT8 = [(0, 1), (2, 3), (4, 5), (6, 7),
         (0, 2), (1, 3), (4, 6), (5, 7),
         (1, 2), (5, 6),
         (0, 4), (1, 5), (2, 6), (3, 7),
         (2, 4), (3, 5),
         (1, 2), (3, 4), (5, 6)]
BITONIC8 = [(0, 4), (1, 5), (2, 6), (3, 7),
            (0, 2), (1, 3), (4, 6), (5, 7),
            (0, 1), (2, 3), (4, 5), (6, 7)]

_mesh = plsc.VectorSubcoreMesh(
    core_axis_name="c", subcore_axis_name="s", num_cores=NC, num_subcores=NS
)


@functools.partial(
    pl.kernel,
    out_type=jax.ShapeDtypeStruct((ROWS, L), jnp.float32),
    mesh=_mesh,
    scratch_types=[
        pltpu.VMEM((2, COLS), jnp.float32),
        pltpu.VMEM((RPW, L), jnp.float32),
        pltpu.SemaphoreType.DMA,
        pltpu.SemaphoreType.DMA,
    ],
    compiler_params=pltpu.CompilerParams(needs_layout_passes=False),
)
def _topk_sc(x_hbm, out_hbm, buf, obuf, sem0, sem1):
    wid = lax.axis_index("s") * NC + lax.axis_index("c")
    base = wid * RPW
    sems = (sem0, sem1)
    copies = [None, None]
    copies[0] = pltpu.async_copy(x_hbm.at[base], buf.at[0], sems[0])
    for r in range(RPW):
        slot = r % 2
        if r + 1 < RPW:
            copies[1 - slot] = pltpu.async_copy(
                x_hbm.at[base + r + 1], buf.at[1 - slot], sems[1 - slot]
            )
        copies[slot].wait()

        ninf = jnp.full((L,), -jnp.inf, dtype=jnp.float32)
        regs = (ninf,) * K

        def body(i, regs, slot=slot):
            regs = list(regs)
            for t in range(BLK_UNROLL):
                off = (i * BLK_UNROLL + t) * (BLK * L)
                s = [buf[slot, pl.ds(off + u * L, L)] for u in range(BLK)]
                for a, b in SORT8:
                    hi = jnp.maximum(s[a], s[b])
                    lo = jnp.minimum(s[a], s[b])
                    s[a], s[b] = hi, lo
                c = [jnp.maximum(regs[j], s[K - 1 - j]) for j in range(K)]
                for a, b in BITONIC8:
                    hi = jnp.maximum(c[a], c[b])
                    lo = jnp.minimum(c[a], c[b])
                    c[a], c[b] = hi, lo
                regs = c
            return tuple(regs)

        regs = lax.fori_loop(0, NBLK // BLK_UNROLL, body, regs)

        best, _ = plsc.sort_key_val(regs[0], regs[0], descending=True)
        for j in range(1, K):
            asc, _ = plsc.sort_key_val(regs[j], regs[j], descending=False)
            best = jnp.maximum(best, asc)
            best, _ = plsc.sort_key_val(best, best, descending=True)
        obuf[r] = best
    pltpu.sync_copy(obuf, out_hbm.at[pl.ds(base, RPW)])


def kernel(x):
    return _topk_sc(x)[:, :K]
